# Initial kernel scaffold; baseline (speedup 1.0000x reference)
import jax
import jax.numpy as jnp
from jax.experimental import pallas as pl


def kernel(x, emb0, emb1, emb2, emb3, emb4, emb5, emb6, emb7, emb8, W, b):
    raise NotImplementedError("write your pallas kernel here")


# speedup vs baseline: 6.2751x; 6.2751x over previous
import functools

import jax
import jax.numpy as jnp
from jax.experimental import pallas as pl

_CAT_DIMS = [119, 5, 12, 12, 10, 6, 6, 2, 2]
_NC = len(_CAT_DIMS)
_OFFS = [0]
for _d in _CAT_DIMS[:-1]:
    _OFFS.append(_OFFS[-1] + _d)
_TOT = sum(_CAT_DIMS)
_KPAD = 256
_NSC = 35
_EMB = 128
_BLOCK = 2000


def _fused_body(x_ref, t_ref, w_ref, b_ref, o_ref):
    xb = x_ref[...]
    cat = xb[:, :_NC].astype(jnp.int32)
    iota = jax.lax.broadcasted_iota(jnp.int32, (1, _KPAD), 1)
    bsz = xb.shape[0]
    acc = jnp.zeros((bsz, _KPAD), jnp.float32)
    for i in range(_NC):
        acc = acc + ((cat[:, i:i + 1] + _OFFS[i]) == iota).astype(jnp.float32)
    emb = jnp.dot(acc, t_ref[...], preferred_element_type=jnp.float32,
                  precision=jax.lax.Precision.HIGHEST)
    lin = jnp.dot(xb[:, _NC:_NC + _NSC], w_ref[...],
                  preferred_element_type=jnp.float32,
                  precision=jax.lax.Precision.HIGHEST)
    o_ref[...] = emb + lin + b_ref[...]


@functools.partial(jax.jit, static_argnames=())
def kernel(x, emb0, emb1, emb2, emb3, emb4, emb5, emb6, emb7, emb8, W, b):
    n, nf = x.shape
    stacked = jnp.concatenate(
        [emb0, emb1, emb2, emb3, emb4, emb5, emb6, emb7, emb8], axis=0)
    tpad = jnp.zeros((_KPAD, _EMB), x.dtype).at[:_TOT].set(stacked)
    b2 = b.reshape(1, _EMB)
    grid = (n // _BLOCK,)
    return pl.pallas_call(
        _fused_body,
        grid=grid,
        in_specs=[
            pl.BlockSpec((_BLOCK, nf), lambda i: (i, 0)),
            pl.BlockSpec((_KPAD, _EMB), lambda i: (0, 0)),
            pl.BlockSpec((_NSC, _EMB), lambda i: (0, 0)),
            pl.BlockSpec((1, _EMB), lambda i: (0, 0)),
        ],
        out_specs=pl.BlockSpec((_BLOCK, _EMB), lambda i: (i, 0)),
        out_shape=jax.ShapeDtypeStruct((n, _EMB), x.dtype),
    )(x, tpad, W, b2)


# speedup vs baseline: 10.0452x; 1.6008x over previous
import functools

import jax
import jax.numpy as jnp
from jax.experimental import pallas as pl

_CAT_DIMS = [119, 5, 12, 12, 10, 6, 6, 2, 2]
_NC = len(_CAT_DIMS)
_OFFS = [0]
for _d in _CAT_DIMS[:-1]:
    _OFFS.append(_OFFS[-1] + _d)
_TOT = sum(_CAT_DIMS)
_KPAD = 256
_NSC = 35
_EMB = 128
_BLOCK = 2000


def _fused_body(x_ref, t_ref, w_ref, b_ref, o_ref):
    xb = x_ref[...]
    cat = xb[:, :_NC].astype(jnp.int32)
    iota = jax.lax.broadcasted_iota(jnp.int32, (1, _KPAD), 1)
    bsz = xb.shape[0]
    sel = jnp.full((bsz, _KPAD), -1, jnp.int32)
    for i in range(_NC):
        seg = (iota >= _OFFS[i]) & (iota < _OFFS[i] + _CAT_DIMS[i])
        sel = jnp.where(seg, cat[:, i:i + 1] + _OFFS[i], sel)
    onehot = (sel == iota).astype(jnp.float32)
    emb = jnp.dot(onehot, t_ref[...], preferred_element_type=jnp.float32)
    lin = jnp.dot(xb[:, _NC:_NC + _NSC], w_ref[...],
                  preferred_element_type=jnp.float32)
    o_ref[...] = emb + lin + b_ref[...]


@functools.partial(jax.jit, static_argnames=())
def kernel(x, emb0, emb1, emb2, emb3, emb4, emb5, emb6, emb7, emb8, W, b):
    n, nf = x.shape
    stacked = jnp.concatenate(
        [emb0, emb1, emb2, emb3, emb4, emb5, emb6, emb7, emb8], axis=0)
    tpad = jnp.zeros((_KPAD, _EMB), x.dtype).at[:_TOT].set(stacked)
    b2 = b.reshape(1, _EMB)
    grid = (n // _BLOCK,)
    return pl.pallas_call(
        _fused_body,
        grid=grid,
        in_specs=[
            pl.BlockSpec((_BLOCK, nf), lambda i: (i, 0)),
            pl.BlockSpec((_KPAD, _EMB), lambda i: (0, 0)),
            pl.BlockSpec((_NSC, _EMB), lambda i: (0, 0)),
            pl.BlockSpec((1, _EMB), lambda i: (0, 0)),
        ],
        out_specs=pl.BlockSpec((_BLOCK, _EMB), lambda i: (i, 0)),
        out_shape=jax.ShapeDtypeStruct((n, _EMB), x.dtype),
    )(x, tpad, W, b2)


# speedup vs baseline: 18.2097x; 1.8128x over previous
import functools

import jax
import jax.numpy as jnp
import numpy as np
from jax.experimental import pallas as pl

_CAT_DIMS = [119, 5, 12, 12, 10, 6, 6, 2, 2]
_NC = len(_CAT_DIMS)
_OFFS = [0]
for _d in _CAT_DIMS[:-1]:
    _OFFS.append(_OFFS[-1] + _d)
_TOT = sum(_CAT_DIMS)
_KPAD = 256
_NSC = 35
_NF = _NC + _NSC
_EMB = 128
_BLOCK = 2000

_SEG = np.full((_KPAD,), -1, np.int64)
for _i in range(_NC):
    _SEG[_OFFS[_i]:_OFFS[_i] + _CAT_DIMS[_i]] = _i

_S_NP = (_SEG[None, :] == np.arange(_NC)[:, None]).astype(np.float32)
_JLOC_NP = np.where(_SEG >= 0, np.arange(_KPAD) - np.array(_OFFS + [0])[
    np.maximum(_SEG, 0)], -5.0).astype(np.float32)


def _fused_body(x_ref, r_ref, t_ref, jl_ref, b_ref, o_ref):
    xb = x_ref[...]
    lane = jax.lax.broadcasted_iota(jnp.int32, (1, _NF), 1)
    xf = jnp.where(lane < _NC, jnp.floor(xb), xb)
    comb = jnp.dot(xf, r_ref[...], preferred_element_type=jnp.float32)
    onehot = (comb[:, :_KPAD] == jl_ref[...]).astype(jnp.float32)
    emb = jnp.dot(onehot, t_ref[...], preferred_element_type=jnp.float32)
    o_ref[...] = emb + comb[:, _KPAD:] + b_ref[...]


@jax.jit
def kernel(x, emb0, emb1, emb2, emb3, emb4, emb5, emb6, emb7, emb8, W, b):
    n, nf = x.shape
    stacked = jnp.concatenate(
        [emb0, emb1, emb2, emb3, emb4, emb5, emb6, emb7, emb8], axis=0)
    tpad = jnp.zeros((_KPAD, _EMB), x.dtype).at[:_TOT].set(stacked)
    rcomb = jnp.zeros((_NF, _KPAD + _EMB), x.dtype)
    rcomb = rcomb.at[:_NC, :_KPAD].set(jnp.asarray(_S_NP))
    rcomb = rcomb.at[_NC:, _KPAD:].set(W)
    jloc = jnp.asarray(_JLOC_NP).reshape(1, _KPAD)
    b2 = b.reshape(1, _EMB)
    grid = (n // _BLOCK,)
    return pl.pallas_call(
        _fused_body,
        grid=grid,
        in_specs=[
            pl.BlockSpec((_BLOCK, nf), lambda i: (i, 0)),
            pl.BlockSpec((_NF, _KPAD + _EMB), lambda i: (0, 0)),
            pl.BlockSpec((_KPAD, _EMB), lambda i: (0, 0)),
            pl.BlockSpec((1, _KPAD), lambda i: (0, 0)),
            pl.BlockSpec((1, _EMB), lambda i: (0, 0)),
        ],
        out_specs=pl.BlockSpec((_BLOCK, _EMB), lambda i: (i, 0)),
        out_shape=jax.ShapeDtypeStruct((n, _EMB), x.dtype),
    )(x, rcomb, tpad, jloc, b2)


# speedup vs baseline: 19.6354x; 1.0783x over previous
import jax
import jax.numpy as jnp
import numpy as np
from jax.experimental import pallas as pl
from jax.experimental.pallas import tpu as pltpu

_CAT_DIMS = [119, 5, 12, 12, 10, 6, 6, 2, 2]
_NC = len(_CAT_DIMS)
_NSC = 35
_NF = _NC + _NSC
_EMB = 128
_KPAD = 256
_BLOCK = 2000

_AOFF = []
_o = 0
for _d in _CAT_DIMS:
    _AOFF.append(_o)
    _o += (_d + 7) // 8 * 8
assert _o <= _KPAD - 8

_SEG = np.full((_KPAD,), -1, np.int64)
for _i in range(_NC):
    _SEG[_AOFF[_i]:_AOFF[_i] + _CAT_DIMS[_i]] = _i

_S_NP = np.zeros((_NF, _KPAD), np.float32)
_S_NP[:_NC] = (_SEG[None, :] == np.arange(_NC)[:, None]).astype(np.float32)

_JLOC_NP = np.full((1, _KPAD), -5.0, np.float32)
for _j in range(_KPAD):
    if _SEG[_j] >= 0:
        _JLOC_NP[0, _j] = _j - _AOFF[_SEG[_j]]
_JLOC_NP[0, _KPAD - 1] = 0.0


def _fused_body(x_ref, e0, e1, e2, e3, e4, e5, e6, e7, e8, w_ref, b_ref,
                s_ref, jl_ref, o_ref, t_scr):
    @pl.when(pl.program_id(0) == 0)
    def _init():
        t_scr[...] = jnp.zeros((_KPAD, _EMB), jnp.float32)
        for eref, aoff, d in zip((e0, e1, e2, e3, e4, e5, e6, e7, e8),
                                 _AOFF, _CAT_DIMS):
            t_scr[aoff:aoff + d, :] = eref[...]
        t_scr[_KPAD - 1:_KPAD, :] = b_ref[...]

    xb = x_ref[...]
    lane = jax.lax.broadcasted_iota(jnp.int32, (1, _NF), 1)
    xf = jnp.where(lane < _NC, jnp.floor(xb), xb)
    sel = jnp.dot(xf, s_ref[...], preferred_element_type=jnp.float32)
    onehot = (sel == jl_ref[...]).astype(jnp.float32)
    emb = jnp.dot(onehot, t_scr[...], preferred_element_type=jnp.float32)
    lin = jnp.dot(xb[:, _NC:], w_ref[...], preferred_element_type=jnp.float32)
    o_ref[...] = emb + lin


@jax.jit
def kernel(x, emb0, emb1, emb2, emb3, emb4, emb5, emb6, emb7, emb8, W, b):
    n, nf = x.shape
    tables = (emb0, emb1, emb2, emb3, emb4, emb5, emb6, emb7, emb8)
    b2 = b.reshape(1, _EMB)
    s_const = jnp.asarray(_S_NP)
    jl_const = jnp.asarray(_JLOC_NP)
    grid = (n // _BLOCK,)
    full = lambda shape: pl.BlockSpec(shape, lambda i: tuple(0 for _ in shape))
    return pl.pallas_call(
        _fused_body,
        grid=grid,
        in_specs=[pl.BlockSpec((_BLOCK, nf), lambda i: (i, 0))]
        + [full(t.shape) for t in tables]
        + [full(W.shape), full((1, _EMB)), full(s_const.shape),
           full((1, _KPAD))],
        out_specs=pl.BlockSpec((_BLOCK, _EMB), lambda i: (i, 0)),
        out_shape=jax.ShapeDtypeStruct((n, _EMB), x.dtype),
        scratch_shapes=[pltpu.VMEM((_KPAD, _EMB), jnp.float32)],
    )(x, *tables, W, b2, s_const, jl_const)


# speedup vs baseline: 23.0522x; 1.1740x over previous
import jax
import jax.numpy as jnp
import numpy as np
from jax.experimental import pallas as pl
from jax.experimental.pallas import tpu as pltpu

_CAT_DIMS = [119, 5, 12, 12, 10, 6, 6, 2, 2]
_NC = len(_CAT_DIMS)
_NSC = 35
_NF = _NC + _NSC
_EMB = 128
_KPAD = 256
_BLOCK = 4000

_AOFF = []
_o = 0
for _d in _CAT_DIMS:
    _AOFF.append(_o)
    _o += (_d + 7) // 8 * 8
assert _o <= _KPAD - 8

_SEG = np.full((_KPAD,), -1, np.int64)
for _i in range(_NC):
    _SEG[_AOFF[_i]:_AOFF[_i] + _CAT_DIMS[_i]] = _i

_S_NP = np.zeros((_NF, _KPAD), np.float32)
_S_NP[:_NC] = (_SEG[None, :] == np.arange(_NC)[:, None]).astype(np.float32)

_JLOC_NP = np.full((1, _KPAD), -5.0, np.float32)
for _j in range(_KPAD):
    if _SEG[_j] >= 0:
        _JLOC_NP[0, _j] = _j - _AOFF[_SEG[_j]]
_JLOC_NP[0, _KPAD - 1] = 0.0


def _fused_body(x_ref, e0, e1, e2, e3, e4, e5, e6, e7, e8, w_ref, b_ref,
                s_ref, jl_ref, o_ref, t_scr):
    @pl.when(pl.program_id(0) == 0)
    def _init():
        t_scr[...] = jnp.zeros((_KPAD, _EMB), jnp.float32)
        for eref, aoff, d in zip((e0, e1, e2, e3, e4, e5, e6, e7, e8),
                                 _AOFF, _CAT_DIMS):
            t_scr[aoff:aoff + d, :] = eref[...]
        t_scr[_KPAD - 1:_KPAD, :] = b_ref[...]

    xb = x_ref[...]
    lane = jax.lax.broadcasted_iota(jnp.int32, (1, _NF), 1)
    xf = jnp.where(lane < _NC, jnp.floor(xb), xb)
    sel = jnp.dot(xf, s_ref[...], preferred_element_type=jnp.float32)
    onehot = (sel == jl_ref[...]).astype(jnp.float32)
    emb = jnp.dot(onehot, t_scr[...], preferred_element_type=jnp.float32)
    lin = jnp.dot(xb[:, _NC:], w_ref[...], preferred_element_type=jnp.float32)
    o_ref[...] = emb + lin


@jax.jit
def kernel(x, emb0, emb1, emb2, emb3, emb4, emb5, emb6, emb7, emb8, W, b):
    n, nf = x.shape
    tables = (emb0, emb1, emb2, emb3, emb4, emb5, emb6, emb7, emb8)
    b2 = b.reshape(1, _EMB)
    s_const = jnp.asarray(_S_NP)
    jl_const = jnp.asarray(_JLOC_NP)
    grid = (n // _BLOCK,)
    full = lambda shape: pl.BlockSpec(shape, lambda i: tuple(0 for _ in shape))
    return pl.pallas_call(
        _fused_body,
        grid=grid,
        in_specs=[pl.BlockSpec((_BLOCK, nf), lambda i: (i, 0))]
        + [full(t.shape) for t in tables]
        + [full(W.shape), full((1, _EMB)), full(s_const.shape),
           full((1, _KPAD))],
        out_specs=pl.BlockSpec((_BLOCK, _EMB), lambda i: (i, 0)),
        out_shape=jax.ShapeDtypeStruct((n, _EMB), x.dtype),
        scratch_shapes=[pltpu.VMEM((_KPAD, _EMB), jnp.float32)],
    )(x, *tables, W, b2, s_const, jl_const)


# speedup vs baseline: 25.5060x; 1.1064x over previous
import jax
import jax.numpy as jnp
import numpy as np
from jax.experimental import pallas as pl
from jax.experimental.pallas import tpu as pltpu

_CAT_DIMS = [119, 5, 12, 12, 10, 6, 6, 2, 2]
_NC = len(_CAT_DIMS)
_NSC = 35
_NF = _NC + _NSC
_EMB = 128
_KPAD = 256
_BLOCK = 10000

_AOFF = []
_o = 0
for _d in _CAT_DIMS:
    _AOFF.append(_o)
    _o += (_d + 7) // 8 * 8
assert _o <= _KPAD - 8

_SEG = np.full((_KPAD,), -1, np.int64)
for _i in range(_NC):
    _SEG[_AOFF[_i]:_AOFF[_i] + _CAT_DIMS[_i]] = _i

_S_NP = np.zeros((_NF, _KPAD), np.float32)
_S_NP[:_NC] = (_SEG[None, :] == np.arange(_NC)[:, None]).astype(np.float32)

_JLOC_NP = np.full((1, _KPAD), -5.0, np.float32)
for _j in range(_KPAD):
    if _SEG[_j] >= 0:
        _JLOC_NP[0, _j] = _j - _AOFF[_SEG[_j]]
_JLOC_NP[0, _KPAD - 1] = 0.0


def _fused_body(x_ref, e0, e1, e2, e3, e4, e5, e6, e7, e8, w_ref, b_ref,
                s_ref, jl_ref, o_ref, t_scr):
    @pl.when(pl.program_id(0) == 0)
    def _init():
        t_scr[...] = jnp.zeros((_KPAD, _EMB), jnp.float32)
        for eref, aoff, d in zip((e0, e1, e2, e3, e4, e5, e6, e7, e8),
                                 _AOFF, _CAT_DIMS):
            t_scr[aoff:aoff + d, :] = eref[...]
        t_scr[_KPAD - 1:_KPAD, :] = b_ref[...]

    xb = x_ref[...]
    lane = jax.lax.broadcasted_iota(jnp.int32, (1, _NF), 1)
    xf = jnp.where(lane < _NC, jnp.floor(xb), xb)
    sel = jnp.dot(xf, s_ref[...], preferred_element_type=jnp.float32)
    onehot = (sel == jl_ref[...]).astype(jnp.float32)
    emb = jnp.dot(onehot, t_scr[...], preferred_element_type=jnp.float32)
    lin = jnp.dot(xb[:, _NC:], w_ref[...], preferred_element_type=jnp.float32)
    o_ref[...] = emb + lin


@jax.jit
def kernel(x, emb0, emb1, emb2, emb3, emb4, emb5, emb6, emb7, emb8, W, b):
    n, nf = x.shape
    tables = (emb0, emb1, emb2, emb3, emb4, emb5, emb6, emb7, emb8)
    b2 = b.reshape(1, _EMB)
    s_const = jnp.asarray(_S_NP)
    jl_const = jnp.asarray(_JLOC_NP)
    grid = (n // _BLOCK,)
    full = lambda shape: pl.BlockSpec(shape, lambda i: tuple(0 for _ in shape))
    return pl.pallas_call(
        _fused_body,
        grid=grid,
        in_specs=[pl.BlockSpec((_BLOCK, nf), lambda i: (i, 0))]
        + [full(t.shape) for t in tables]
        + [full(W.shape), full((1, _EMB)), full(s_const.shape),
           full((1, _KPAD))],
        out_specs=pl.BlockSpec((_BLOCK, _EMB), lambda i: (i, 0)),
        out_shape=jax.ShapeDtypeStruct((n, _EMB), x.dtype),
        scratch_shapes=[pltpu.VMEM((_KPAD, _EMB), jnp.float32)],
    )(x, *tables, W, b2, s_const, jl_const)


# speedup vs baseline: 25.6117x; 1.0041x over previous
import jax
import jax.numpy as jnp
import numpy as np
from jax.experimental import pallas as pl
from jax.experimental.pallas import tpu as pltpu

_CAT_DIMS = [119, 5, 12, 12, 10, 6, 6, 2, 2]
_NC = len(_CAT_DIMS)
_NSC = 35
_NF = _NC + _NSC
_EMB = 128
_KPAD = 256
_BLOCK = 20000

_AOFF = []
_o = 0
for _d in _CAT_DIMS:
    _AOFF.append(_o)
    _o += (_d + 7) // 8 * 8
assert _o <= _KPAD - 8

_SEG = np.full((_KPAD,), -1, np.int64)
for _i in range(_NC):
    _SEG[_AOFF[_i]:_AOFF[_i] + _CAT_DIMS[_i]] = _i

_S_NP = np.zeros((_NF, _KPAD), np.float32)
_S_NP[:_NC] = (_SEG[None, :] == np.arange(_NC)[:, None]).astype(np.float32)

_JLOC_NP = np.full((1, _KPAD), -5.0, np.float32)
for _j in range(_KPAD):
    if _SEG[_j] >= 0:
        _JLOC_NP[0, _j] = _j - _AOFF[_SEG[_j]]
_JLOC_NP[0, _KPAD - 1] = 0.0


def _fused_body(x_ref, e0, e1, e2, e3, e4, e5, e6, e7, e8, w_ref, b_ref,
                s_ref, jl_ref, o_ref, t_scr):
    @pl.when(pl.program_id(0) == 0)
    def _init():
        t_scr[...] = jnp.zeros((_KPAD, _EMB), jnp.float32)
        for eref, aoff, d in zip((e0, e1, e2, e3, e4, e5, e6, e7, e8),
                                 _AOFF, _CAT_DIMS):
            t_scr[aoff:aoff + d, :] = eref[...]
        t_scr[_KPAD - 1:_KPAD, :] = b_ref[...]

    xb = x_ref[...]
    lane = jax.lax.broadcasted_iota(jnp.int32, (1, _NF), 1)
    xf = jnp.where(lane < _NC, jnp.floor(xb), xb)
    sel = jnp.dot(xf, s_ref[...], preferred_element_type=jnp.float32)
    onehot = (sel == jl_ref[...]).astype(jnp.float32)
    emb = jnp.dot(onehot, t_scr[...], preferred_element_type=jnp.float32)
    lin = jnp.dot(xb[:, _NC:], w_ref[...], preferred_element_type=jnp.float32)
    o_ref[...] = emb + lin


@jax.jit
def kernel(x, emb0, emb1, emb2, emb3, emb4, emb5, emb6, emb7, emb8, W, b):
    n, nf = x.shape
    tables = (emb0, emb1, emb2, emb3, emb4, emb5, emb6, emb7, emb8)
    b2 = b.reshape(1, _EMB)
    s_const = jnp.asarray(_S_NP)
    jl_const = jnp.asarray(_JLOC_NP)
    grid = (n // _BLOCK,)
    full = lambda shape: pl.BlockSpec(shape, lambda i: tuple(0 for _ in shape))
    return pl.pallas_call(
        _fused_body,
        grid=grid,
        in_specs=[pl.BlockSpec((_BLOCK, nf), lambda i: (i, 0))]
        + [full(t.shape) for t in tables]
        + [full(W.shape), full((1, _EMB)), full(s_const.shape),
           full((1, _KPAD))],
        out_specs=pl.BlockSpec((_BLOCK, _EMB), lambda i: (i, 0)),
        out_shape=jax.ShapeDtypeStruct((n, _EMB), x.dtype),
        scratch_shapes=[pltpu.VMEM((_KPAD, _EMB), jnp.float32)],
    )(x, *tables, W, b2, s_const, jl_const)
